# Initial kernel scaffold; baseline (speedup 1.0000x reference)
import jax
import jax.numpy as jnp
from jax.experimental import pallas as pl


def kernel(center, target, negative, W_center, W_context):
    raise NotImplementedError("write your pallas kernel here")


# speedup vs baseline: 4.1313x; 4.1313x over previous
import functools

import jax
import jax.numpy as jnp
from jax import lax
from jax.experimental import pallas as pl
from jax.experimental.pallas import tpu as pltpu
from jax.experimental.pallas import tpu_sc as plsc

_VOCAB = 1000000
_DIM = 64
_B = 16384
_K = 20
_NW = 32
_BPW = _B // _NW
_NG = _BPW // 16
_ROWS = 16 * _K


def _sc_body(center_h, target_h, negf_h, wcen_h, wctx_h, out_h,
             cen_i, tgt_i, neg_i,
             ci_a, ti_a, n0_a, n1_a, n2_a,
             ci_b, ti_b, n0_b, n1_b, n2_b,
             cr_a, tr_a, nr_a, cr_b, tr_b, nr_b,
             dots_v, sem_a, sem_b):
    wid = lax.axis_index("s") * 2 + lax.axis_index("c")
    b0 = pl.multiple_of(wid * _BPW, _BPW)
    pltpu.sync_copy(center_h.at[pl.ds(b0, _BPW)], cen_i)
    pltpu.sync_copy(target_h.at[pl.ds(b0, _BPW)], tgt_i)
    pltpu.sync_copy(negf_h.at[pl.ds(b0 * _K, _BPW * _K)], neg_i)

    iota = lax.broadcasted_iota(jnp.int32, (16,), 0)
    nrows = [iota * _K + k for k in range(_K)]

    def copies(bufs, sem):
        ci, ti, n0, n1, n2, cr, tr, nr = bufs
        return [
            pltpu.make_async_copy(wcen_h.at[ci], cr, sem),
            pltpu.make_async_copy(wctx_h.at[ti], tr, sem),
            pltpu.make_async_copy(wctx_h.at[n0], nr.at[pl.ds(0, 128)], sem),
            pltpu.make_async_copy(wctx_h.at[n1], nr.at[pl.ds(128, 128)], sem),
            pltpu.make_async_copy(wctx_h.at[n2], nr.at[pl.ds(256, 64)], sem),
        ]

    def fire(bg, bufs, sem):
        ci, ti, n0, n1, n2, cr, tr, nr = bufs
        i16 = pl.multiple_of(bg * 16, 16)
        r0 = pl.multiple_of(bg * _ROWS, 64)
        ci[...] = cen_i[pl.ds(i16, 16)]
        ti[...] = tgt_i[pl.ds(i16, 16)]
        for j in range(8):
            n0[pl.ds(16 * j, 16)] = neg_i[pl.ds(r0 + 16 * j, 16)]
        for j in range(8):
            n1[pl.ds(16 * j, 16)] = neg_i[pl.ds(r0 + 128 + 16 * j, 16)]
        for j in range(4):
            n2[pl.ds(16 * j, 16)] = neg_i[pl.ds(r0 + 256 + 16 * j, 16)]
        for cp in copies(bufs, sem):
            cp.start()

    def drain(bufs, sem):
        for cp in copies(bufs, sem):
            cp.wait()

    def compute(bg, cr, tr, nr):
        def dstep(d, accs):
            dcol = jnp.full((16,), d, jnp.int32)
            c = plsc.load_gather(cr, [iota, dcol])
            t = plsc.load_gather(tr, [iota, dcol])
            new = [accs[0] - c * t]
            for k in range(_K):
                nv = plsc.load_gather(nr, [nrows[k], dcol])
                new.append(accs[k + 1] + nv * c)
            return tuple(new)

        accs = lax.fori_loop(
            0, _DIM, dstep,
            tuple(jnp.zeros((16,), jnp.float32) for _ in range(_K + 1)))
        o16 = pl.multiple_of(bg * 16, 16)
        for j in range(_K + 1):
            dots_v[j, pl.ds(o16, 16)] = accs[j]

    bufs_a = (ci_a, ti_a, n0_a, n1_a, n2_a, cr_a, tr_a, nr_a)
    bufs_b = (ci_b, ti_b, n0_b, n1_b, n2_b, cr_b, tr_b, nr_b)
    fire(0, bufs_a, sem_a)

    def outer(i, carry):
        bg = i * 2
        fire(bg + 1, bufs_b, sem_b)
        drain(bufs_a, sem_a)
        compute(bg, cr_a, tr_a, nr_a)

        @pl.when(bg + 2 < _NG)
        def _():
            fire(bg + 2, bufs_a, sem_a)

        drain(bufs_b, sem_b)
        compute(bg + 1, cr_b, tr_b, nr_b)
        return carry

    lax.fori_loop(0, _NG // 2, outer, 0)
    pltpu.sync_copy(dots_v, out_h.at[wid])


@jax.jit
def _sc_dots(center, target, neg_flat, w_center, w_context):
    mesh = plsc.VectorSubcoreMesh(core_axis_name="c", subcore_axis_name="s")
    return pl.kernel(
        _sc_body,
        out_type=jax.ShapeDtypeStruct((_NW, _K + 1, _BPW), jnp.float32),
        mesh=mesh,
        compiler_params=pltpu.CompilerParams(
            needs_layout_passes=False, use_tc_tiling_on_sc=False),
        scratch_types=[
            pltpu.VMEM((_BPW,), jnp.int32),
            pltpu.VMEM((_BPW,), jnp.int32),
            pltpu.VMEM((_BPW * _K,), jnp.int32),
            pltpu.VMEM((16,), jnp.int32),
            pltpu.VMEM((16,), jnp.int32),
            pltpu.VMEM((128,), jnp.int32),
            pltpu.VMEM((128,), jnp.int32),
            pltpu.VMEM((64,), jnp.int32),
            pltpu.VMEM((16,), jnp.int32),
            pltpu.VMEM((16,), jnp.int32),
            pltpu.VMEM((128,), jnp.int32),
            pltpu.VMEM((128,), jnp.int32),
            pltpu.VMEM((64,), jnp.int32),
            pltpu.VMEM((16, _DIM), jnp.float32),
            pltpu.VMEM((16, _DIM), jnp.float32),
            pltpu.VMEM((_ROWS, _DIM), jnp.float32),
            pltpu.VMEM((16, _DIM), jnp.float32),
            pltpu.VMEM((16, _DIM), jnp.float32),
            pltpu.VMEM((_ROWS, _DIM), jnp.float32),
            pltpu.VMEM((_K + 1, _BPW), jnp.float32),
            pltpu.SemaphoreType.DMA,
            pltpu.SemaphoreType.DMA,
        ],
    )(center, target, neg_flat, w_center, w_context)


def _loss_body(x_ref, o_ref):
    x = x_ref[...]
    ls = jnp.minimum(x, 0.0) - jnp.log1p(jnp.exp(-jnp.abs(x)))
    o_ref[0, 0] = -(jnp.sum(ls) + (_K - 1) * jnp.sum(ls[:, 0, :]))


def kernel(center, target, negative, W_center, W_context):
    center = center.astype(jnp.int32)
    target = target.astype(jnp.int32)
    neg_flat = negative.astype(jnp.int32).reshape(-1)
    dots = _sc_dots(center, target, neg_flat, W_center, W_context)
    loss = pl.pallas_call(
        _loss_body,
        out_shape=jax.ShapeDtypeStruct((1, 1), jnp.float32),
        out_specs=pl.BlockSpec(memory_space=pltpu.SMEM),
    )(dots)
    return loss[0, 0]


# speedup vs baseline: 4.1431x; 1.0029x over previous
import functools

import jax
import jax.numpy as jnp
from jax import lax
from jax.experimental import pallas as pl
from jax.experimental.pallas import tpu as pltpu
from jax.experimental.pallas import tpu_sc as plsc

_VOCAB = 1000000
_DIM = 64
_B = 16384
_K = 20
_NW = 32
_BPW = _B // _NW
_NG = _BPW // 16
_ROWS = 16 * _K


def _sc_body(center_h, target_h, negf_h, wcen_h, wctx_h, out_h,
             cen_i, tgt_i, neg_i,
             ci_a, ti_a, n0_a, n1_a, n2_a,
             ci_b, ti_b, n0_b, n1_b, n2_b,
             cr_a, tr_a, nr_a, cr_b, tr_b, nr_b,
             dots_v, sem_a, sem_b):
    wid = lax.axis_index("s") * 2 + lax.axis_index("c")
    b0 = pl.multiple_of(wid * _BPW, _BPW)
    pltpu.sync_copy(center_h.at[pl.ds(b0, _BPW)], cen_i)
    pltpu.sync_copy(target_h.at[pl.ds(b0, _BPW)], tgt_i)
    pltpu.sync_copy(negf_h.at[pl.ds(b0 * _K, _BPW * _K)], neg_i)

    iota = lax.broadcasted_iota(jnp.int32, (16,), 0)
    nrows = [iota * _K + k for k in range(_K)]
    one = jnp.full((16,), 1, jnp.int32)

    def copies(bufs, sem):
        ci, ti, n0, n1, n2, cr, tr, nr = bufs
        return [
            pltpu.make_async_copy(wcen_h.at[ci], cr, sem),
            pltpu.make_async_copy(wctx_h.at[ti], tr, sem),
            pltpu.make_async_copy(wctx_h.at[n0], nr.at[pl.ds(0, 128)], sem),
            pltpu.make_async_copy(wctx_h.at[n1], nr.at[pl.ds(128, 128)], sem),
            pltpu.make_async_copy(wctx_h.at[n2], nr.at[pl.ds(256, 64)], sem),
        ]

    def fire(bg, bufs, sem):
        ci, ti, n0, n1, n2, cr, tr, nr = bufs
        i16 = pl.multiple_of(bg * 16, 16)
        r0 = pl.multiple_of(bg * _ROWS, 64)
        ci[...] = jnp.right_shift(cen_i[pl.ds(i16, 16)], one)
        ti[...] = jnp.right_shift(tgt_i[pl.ds(i16, 16)], one)
        for j in range(8):
            n0[pl.ds(16 * j, 16)] = jnp.right_shift(
                neg_i[pl.ds(r0 + 16 * j, 16)], one)
        for j in range(8):
            n1[pl.ds(16 * j, 16)] = jnp.right_shift(
                neg_i[pl.ds(r0 + 128 + 16 * j, 16)], one)
        for j in range(4):
            n2[pl.ds(16 * j, 16)] = jnp.right_shift(
                neg_i[pl.ds(r0 + 256 + 16 * j, 16)], one)
        for cp in copies(bufs, sem):
            cp.start()

    def drain(bufs, sem):
        for cp in copies(bufs, sem):
            cp.wait()

    def compute(bg, cr, tr, nr):
        i16 = pl.multiple_of(bg * 16, 16)
        r0 = pl.multiple_of(bg * _ROWS, 64)
        cbase = jnp.left_shift(
            jnp.bitwise_and(cen_i[pl.ds(i16, 16)], one), jnp.full((16,), 6, jnp.int32))
        tbase = jnp.left_shift(
            jnp.bitwise_and(tgt_i[pl.ds(i16, 16)], one), jnp.full((16,), 6, jnp.int32))
        nbase = [
            jnp.left_shift(
                jnp.bitwise_and(plsc.load_gather(neg_i, [r0 + nrows[k]]), one),
                jnp.full((16,), 6, jnp.int32))
            for k in range(_K)
        ]

        def dstep(d, accs):
            dvec = jnp.full((16,), d, jnp.int32)
            c = plsc.load_gather(cr, [iota, cbase + dvec])
            t = plsc.load_gather(tr, [iota, tbase + dvec])
            new = [accs[0] - c * t]
            for k in range(_K):
                nv = plsc.load_gather(nr, [nrows[k], nbase[k] + dvec])
                new.append(accs[k + 1] + nv * c)
            return tuple(new)

        accs = lax.fori_loop(
            0, _DIM, dstep,
            tuple(jnp.zeros((16,), jnp.float32) for _ in range(_K + 1)))
        o16 = pl.multiple_of(bg * 16, 16)
        for j in range(_K + 1):
            dots_v[j, pl.ds(o16, 16)] = accs[j]

    bufs_a = (ci_a, ti_a, n0_a, n1_a, n2_a, cr_a, tr_a, nr_a)
    bufs_b = (ci_b, ti_b, n0_b, n1_b, n2_b, cr_b, tr_b, nr_b)
    fire(0, bufs_a, sem_a)

    def outer(i, carry):
        bg = i * 2
        fire(bg + 1, bufs_b, sem_b)
        drain(bufs_a, sem_a)
        compute(bg, cr_a, tr_a, nr_a)

        @pl.when(bg + 2 < _NG)
        def _():
            fire(bg + 2, bufs_a, sem_a)

        drain(bufs_b, sem_b)
        compute(bg + 1, cr_b, tr_b, nr_b)
        return carry

    lax.fori_loop(0, _NG // 2, outer, 0)
    pltpu.sync_copy(dots_v, out_h.at[wid])


@jax.jit
def _sc_dots(center, target, neg_flat, w_center, w_context):
    mesh = plsc.VectorSubcoreMesh(core_axis_name="c", subcore_axis_name="s")
    return pl.kernel(
        _sc_body,
        out_type=jax.ShapeDtypeStruct((_NW, _K + 1, _BPW), jnp.float32),
        mesh=mesh,
        compiler_params=pltpu.CompilerParams(needs_layout_passes=False),
        scratch_types=[
            pltpu.VMEM((_BPW,), jnp.int32),
            pltpu.VMEM((_BPW,), jnp.int32),
            pltpu.VMEM((_BPW * _K,), jnp.int32),
            pltpu.VMEM((16,), jnp.int32),
            pltpu.VMEM((16,), jnp.int32),
            pltpu.VMEM((128,), jnp.int32),
            pltpu.VMEM((128,), jnp.int32),
            pltpu.VMEM((64,), jnp.int32),
            pltpu.VMEM((16,), jnp.int32),
            pltpu.VMEM((16,), jnp.int32),
            pltpu.VMEM((128,), jnp.int32),
            pltpu.VMEM((128,), jnp.int32),
            pltpu.VMEM((64,), jnp.int32),
            pltpu.VMEM((16, 2 * _DIM), jnp.float32),
            pltpu.VMEM((16, 2 * _DIM), jnp.float32),
            pltpu.VMEM((_ROWS, 2 * _DIM), jnp.float32),
            pltpu.VMEM((16, 2 * _DIM), jnp.float32),
            pltpu.VMEM((16, 2 * _DIM), jnp.float32),
            pltpu.VMEM((_ROWS, 2 * _DIM), jnp.float32),
            pltpu.VMEM((_K + 1, _BPW), jnp.float32),
            pltpu.SemaphoreType.DMA,
            pltpu.SemaphoreType.DMA,
        ],
    )(center, target, neg_flat, w_center, w_context)


def _loss_body(x_ref, o_ref):
    x = x_ref[...]
    ls = jnp.minimum(x, 0.0) - jnp.log1p(jnp.exp(-jnp.abs(x)))
    o_ref[0, 0] = -(jnp.sum(ls) + (_K - 1) * jnp.sum(ls[:, 0, :]))


def kernel(center, target, negative, W_center, W_context):
    center = center.astype(jnp.int32)
    target = target.astype(jnp.int32)
    neg_flat = negative.astype(jnp.int32).reshape(-1)
    w_cen = W_center.reshape(_VOCAB // 2, 2 * _DIM)
    w_ctx = W_context.reshape(_VOCAB // 2, 2 * _DIM)
    dots = _sc_dots(center, target, neg_flat, w_cen, w_ctx)
    loss = pl.pallas_call(
        _loss_body,
        out_shape=jax.ShapeDtypeStruct((1, 1), jnp.float32),
        out_specs=pl.BlockSpec(memory_space=pltpu.SMEM),
    )(dots)
    return loss[0, 0]
